# Initial kernel scaffold; baseline (speedup 1.0000x reference)
import jax
import jax.numpy as jnp
from jax.experimental import pallas as pl


def kernel(x, edge_index, W1, W2):
    raise NotImplementedError("write your pallas kernel here")


# speedup vs baseline: 17.8594x; 17.8594x over previous
import functools

import jax
import jax.numpy as jnp
from jax import lax
from jax.experimental import pallas as pl
from jax.experimental.pallas import tpu as pltpu
from jax.experimental.pallas import tpu_sc as plsc

N_NODES = 10000
HID = 8
NC = 2
NS = 16
NW = NC * NS
CH = 128
NROWS = 10240
RPS = NROWS // NS
DUMMY = N_NODES


def _sc_degrees(src2d, dst2d, zeros1, ones):
    ncv = src2d.shape[0] // NW
    mesh = plsc.VectorSubcoreMesh(core_axis_name="c", subcore_axis_name="s")

    @functools.partial(
        pl.kernel,
        mesh=mesh,
        out_type=(
            jax.ShapeDtypeStruct((NC * NROWS,), jnp.float32),
            jax.ShapeDtypeStruct((NC * NROWS,), jnp.float32),
        ),
        scratch_types=[
            pltpu.VMEM((ncv, CH), jnp.int32),
            pltpu.VMEM((ncv, CH), jnp.int32),
            pltpu.VMEM((CH,), jnp.float32),
            pltpu.VMEM_SHARED((NROWS,), jnp.float32),
            pltpu.VMEM_SHARED((NROWS,), jnp.float32),
        ],
    )
    def k(src_h, dst_h, z_h, ones_h, do_h, di_h, srcv, dstv, onesv, degA, degB):
        c = lax.axis_index("c")
        s = lax.axis_index("s")
        wid = c * NS + s
        pltpu.sync_copy(z_h.at[pl.ds(s * RPS, RPS)], degA.at[pl.ds(s * RPS, RPS)])
        pltpu.sync_copy(z_h.at[pl.ds(s * RPS, RPS)], degB.at[pl.ds(s * RPS, RPS)])
        pltpu.sync_copy(src_h.at[pl.ds(wid * ncv, ncv)], srcv)
        pltpu.sync_copy(dst_h.at[pl.ds(wid * ncv, ncv)], dstv)
        pltpu.sync_copy(ones_h, onesv)
        plsc.subcore_barrier()

        def body(j, carry):
            pltpu.sync_copy(onesv, degA.at[srcv.at[j]], add=True)
            pltpu.sync_copy(onesv, degB.at[dstv.at[j]], add=True)
            return carry

        lax.fori_loop(0, ncv, body, 0)
        plsc.subcore_barrier()
        pltpu.sync_copy(degA.at[pl.ds(s * RPS, RPS)],
                        do_h.at[pl.ds(c * NROWS + s * RPS, RPS)])
        pltpu.sync_copy(degB.at[pl.ds(s * RPS, RPS)],
                        di_h.at[pl.ds(c * NROWS + s * RPS, RPS)])

    return k(src2d, dst2d, zeros1, ones)


def _sc_msgpass(table, src2d, dst2d, zeros2):
    ncv = src2d.shape[0] // NW
    mesh = plsc.VectorSubcoreMesh(core_axis_name="c", subcore_axis_name="s")

    @functools.partial(
        pl.kernel,
        mesh=mesh,
        out_type=jax.ShapeDtypeStruct((NC * NROWS, HID), jnp.float32),
        scratch_types=[
            pltpu.VMEM((ncv, CH), jnp.int32),
            pltpu.VMEM((ncv, CH), jnp.int32),
            pltpu.VMEM((CH, HID), jnp.float32),
            pltpu.VMEM_SHARED((NROWS, HID), jnp.float32),
            pltpu.SemaphoreType.DMA,
        ],
        compiler_params=pltpu.CompilerParams(use_tc_tiling_on_sc=False),
    )
    def k(tab_h, src_h, dst_h, z_h, agg_h, srcv, dstv, rowsv, agg, sem):
        c = lax.axis_index("c")
        s = lax.axis_index("s")
        wid = c * NS + s
        pltpu.sync_copy(z_h.at[pl.ds(s * RPS, RPS)], agg.at[pl.ds(s * RPS, RPS)])
        pltpu.sync_copy(src_h.at[pl.ds(wid * ncv, ncv)], srcv)
        pltpu.sync_copy(dst_h.at[pl.ds(wid * ncv, ncv)], dstv)
        plsc.subcore_barrier()

        def body(j, carry):
            pltpu.async_copy(tab_h.at[srcv.at[j]], rowsv, sem).wait()
            pltpu.sync_copy(rowsv, agg.at[dstv.at[j]], add=True)
            return carry

        lax.fori_loop(0, ncv, body, 0)
        plsc.subcore_barrier()
        pltpu.sync_copy(agg.at[pl.ds(s * RPS, RPS)],
                        agg_h.at[pl.ds(c * NROWS + s * RPS, RPS)])

    return k(table, src2d, dst2d, zeros2)


def _tc_prep(xp, W1, degp_o, degp_i):
    def body(x_r, w_r, do_r, di_r, h_r, no_r, ni_r):
        no = lax.rsqrt(do_r[0, :] + do_r[1, :] + 1.0)
        ni = lax.rsqrt(di_r[0, :] + di_r[1, :] + 1.0)
        no_r[...] = no[:, None]
        ni_r[...] = ni[:, None]
        h_r[...] = jnp.dot(x_r[...] * no[:, None], w_r[...],
                           preferred_element_type=jnp.float32)

    return pl.pallas_call(
        body,
        out_shape=(
            jax.ShapeDtypeStruct((NROWS, HID), jnp.float32),
            jax.ShapeDtypeStruct((NROWS, 1), jnp.float32),
            jax.ShapeDtypeStruct((NROWS, 1), jnp.float32),
        ),
    )(xp, W1, degp_o, degp_i)


def _tc_mid(aggp, h0s, ni, no, W2):
    def body(a_r, h_r, ni_r, no_r, w_r, t_r):
        h1 = (a_r[0] + a_r[1] + h_r[...]) * ni_r[...]
        t_r[...] = jnp.dot(h1, w_r[...],
                           preferred_element_type=jnp.float32) * no_r[...]

    return pl.pallas_call(
        body,
        out_shape=jax.ShapeDtypeStruct((NROWS, HID), jnp.float32),
    )(aggp, h0s, ni, no, W2)


def _tc_fin(aggp, t, ni):
    def body(a_r, t_r, ni_r, o_r):
        o_r[...] = (a_r[0] + a_r[1] + t_r[...]) * ni_r[...]

    return pl.pallas_call(
        body,
        out_shape=jax.ShapeDtypeStruct((NROWS, HID), jnp.float32),
    )(aggp, t, ni)


def kernel(x, edge_index, W1, W2):
    e = edge_index.shape[1]
    ncv = -(-e // (NW * CH))
    ncv = -(-ncv // 8) * 8
    e_pad = NW * ncv * CH
    pad = jnp.full((e_pad - e,), DUMMY, jnp.int32)
    src2d = jnp.concatenate([edge_index[0], pad]).reshape(e_pad // CH, CH)
    dst2d = jnp.concatenate([edge_index[1], pad]).reshape(e_pad // CH, CH)
    xp = jnp.pad(x, ((0, NROWS - x.shape[0]), (0, 0)))
    z1 = jnp.zeros((NROWS,), jnp.float32)
    z2 = jnp.zeros((NROWS, HID), jnp.float32)
    ones = jnp.ones((CH,), jnp.float32)

    degp_o, degp_i = _sc_degrees(src2d, dst2d, z1, ones)
    degp_o = degp_o.reshape(NC, NROWS)
    degp_i = degp_i.reshape(NC, NROWS)
    h0s, no, ni = _tc_prep(xp, W1, degp_o, degp_i)
    aggp1 = _sc_msgpass(h0s, src2d, dst2d, z2).reshape(NC, NROWS, HID)
    t = _tc_mid(aggp1, h0s, ni, no, W2)
    aggp2 = _sc_msgpass(t, src2d, dst2d, z2).reshape(NC, NROWS, HID)
    out = _tc_fin(aggp2, t, ni)
    return out[:N_NODES]


# speedup vs baseline: 22.3206x; 1.2498x over previous
import functools

import jax
import jax.numpy as jnp
from jax import lax
from jax.experimental import pallas as pl
from jax.experimental.pallas import tpu as pltpu
from jax.experimental.pallas import tpu_sc as plsc

N_NODES = 10000
HID = 8
NC = 2
NS = 16
NW = NC * NS
CH = 128
NROWS = 10240
RPS = NROWS // NS
DUMMY = N_NODES


def _sc_degrees(src2d, dst2d, zeros1, ones):
    ncv = src2d.shape[0] // NW
    mesh = plsc.VectorSubcoreMesh(core_axis_name="c", subcore_axis_name="s")

    @functools.partial(
        pl.kernel,
        mesh=mesh,
        out_type=(
            jax.ShapeDtypeStruct((NC * NROWS,), jnp.float32),
            jax.ShapeDtypeStruct((NC * NROWS,), jnp.float32),
        ),
        scratch_types=[
            pltpu.VMEM((ncv, CH), jnp.int32),
            pltpu.VMEM((ncv, CH), jnp.int32),
            pltpu.VMEM((CH,), jnp.float32),
            pltpu.VMEM_SHARED((NROWS,), jnp.float32),
            pltpu.VMEM_SHARED((NROWS,), jnp.float32),
            pltpu.SemaphoreType.DMA,
        ],
    )
    def k(src_h, dst_h, z_h, ones_h, do_h, di_h, srcv, dstv, onesv, degA, degB,
          dsem):
        c = lax.axis_index("c")
        s = lax.axis_index("s")
        wid = c * NS + s
        pltpu.sync_copy(z_h.at[pl.ds(s * RPS, RPS)], degA.at[pl.ds(s * RPS, RPS)])
        pltpu.sync_copy(z_h.at[pl.ds(s * RPS, RPS)], degB.at[pl.ds(s * RPS, RPS)])
        pltpu.sync_copy(src_h.at[pl.ds(wid * ncv, ncv)], srcv)
        pltpu.sync_copy(dst_h.at[pl.ds(wid * ncv, ncv)], dstv)
        pltpu.sync_copy(ones_h, onesv)
        plsc.subcore_barrier()

        def body(j, carry):
            pltpu.async_copy(onesv, degA.at[srcv.at[j]], dsem, add=True)
            pltpu.async_copy(onesv, degB.at[dstv.at[j]], dsem, add=True)
            return carry

        lax.fori_loop(0, ncv, body, 0)

        def drain(j, carry):
            pltpu.make_async_copy(onesv, degA.at[srcv.at[j]], dsem).wait()
            pltpu.make_async_copy(onesv, degB.at[dstv.at[j]], dsem).wait()
            return carry

        lax.fori_loop(0, ncv, drain, 0)
        plsc.subcore_barrier()
        pltpu.sync_copy(degA.at[pl.ds(s * RPS, RPS)],
                        do_h.at[pl.ds(c * NROWS + s * RPS, RPS)])
        pltpu.sync_copy(degB.at[pl.ds(s * RPS, RPS)],
                        di_h.at[pl.ds(c * NROWS + s * RPS, RPS)])

    return k(src2d, dst2d, zeros1, ones)


K_GRP = 5
N_BUF = 2 * K_GRP


def _sc_msgpass(table, src2d, dst2d, zeros2):
    ncv = src2d.shape[0] // NW
    pairs = ncv // (2 * K_GRP)
    assert ncv == pairs * 2 * K_GRP
    mesh = plsc.VectorSubcoreMesh(core_axis_name="c", subcore_axis_name="s")

    @functools.partial(
        pl.kernel,
        mesh=mesh,
        out_type=jax.ShapeDtypeStruct((NC * NROWS, HID), jnp.float32),
        scratch_types=[
            pltpu.VMEM((ncv, CH), jnp.int32),
            pltpu.VMEM((ncv, CH), jnp.int32),
            pltpu.VMEM((N_BUF, CH, HID), jnp.float32),
            pltpu.VMEM_SHARED((NROWS, HID), jnp.float32),
            pltpu.SemaphoreType.DMA,
            pltpu.SemaphoreType.DMA,
            pltpu.SemaphoreType.DMA,
            pltpu.SemaphoreType.DMA,
        ],
        compiler_params=pltpu.CompilerParams(use_tc_tiling_on_sc=False),
    )
    def k(tab_h, src_h, dst_h, z_h, agg_h, srcv, dstv, rowsv, agg,
          gsA, gsB, ssA, ssB):
        c = lax.axis_index("c")
        s = lax.axis_index("s")
        wid = c * NS + s
        pltpu.sync_copy(z_h.at[pl.ds(s * RPS, RPS)], agg.at[pl.ds(s * RPS, RPS)])
        pltpu.sync_copy(src_h.at[pl.ds(wid * ncv, ncv)], srcv)
        pltpu.sync_copy(dst_h.at[pl.ds(wid * ncv, ncv)], dstv)
        plsc.subcore_barrier()

        def pair(p, carry):
            for par, gsem, ssem in ((0, gsA, ssA), (1, gsB, ssB)):
                o = 2 * p + par

                @pl.when(p >= 1)
                def _drain_old():
                    for b in range(K_GRP):
                        g_old = (o - 2) * K_GRP + b
                        pltpu.make_async_copy(
                            rowsv.at[par * K_GRP + b],
                            agg.at[dstv.at[g_old]], ssem).wait()

                for b in range(K_GRP):
                    g = o * K_GRP + b
                    pltpu.async_copy(tab_h.at[srcv.at[g]],
                                     rowsv.at[par * K_GRP + b], gsem)
                for b in range(K_GRP):
                    g = o * K_GRP + b
                    pltpu.make_async_copy(tab_h.at[srcv.at[g]],
                                          rowsv.at[par * K_GRP + b], gsem).wait()
                for b in range(K_GRP):
                    g = o * K_GRP + b
                    pltpu.async_copy(rowsv.at[par * K_GRP + b],
                                     agg.at[dstv.at[g]], ssem, add=True)
            return carry

        lax.fori_loop(0, pairs, pair, 0)
        for par, ssem in ((0, ssA), (1, ssB)):
            o = (pairs - 1) * 2 + par
            for b in range(K_GRP):
                g = o * K_GRP + b
                pltpu.make_async_copy(rowsv.at[par * K_GRP + b],
                                      agg.at[dstv.at[g]], ssem).wait()
        plsc.subcore_barrier()
        pltpu.sync_copy(agg.at[pl.ds(s * RPS, RPS)],
                        agg_h.at[pl.ds(c * NROWS + s * RPS, RPS)])

    return k(table, src2d, dst2d, zeros2)


def _tc_prep(xp, W1, degp_o, degp_i):
    def body(x_r, w_r, do_r, di_r, h_r, no_r, ni_r):
        no = lax.rsqrt(do_r[0, :] + do_r[1, :] + 1.0)
        ni = lax.rsqrt(di_r[0, :] + di_r[1, :] + 1.0)
        no_r[...] = no[:, None]
        ni_r[...] = ni[:, None]
        h_r[...] = jnp.dot(x_r[...] * no[:, None], w_r[...],
                           preferred_element_type=jnp.float32)

    return pl.pallas_call(
        body,
        out_shape=(
            jax.ShapeDtypeStruct((NROWS, HID), jnp.float32),
            jax.ShapeDtypeStruct((NROWS, 1), jnp.float32),
            jax.ShapeDtypeStruct((NROWS, 1), jnp.float32),
        ),
    )(xp, W1, degp_o, degp_i)


def _tc_mid(aggp, h0s, ni, no, W2):
    def body(a_r, h_r, ni_r, no_r, w_r, t_r):
        h1 = (a_r[0] + a_r[1] + h_r[...]) * ni_r[...]
        t_r[...] = jnp.dot(h1, w_r[...],
                           preferred_element_type=jnp.float32) * no_r[...]

    return pl.pallas_call(
        body,
        out_shape=jax.ShapeDtypeStruct((NROWS, HID), jnp.float32),
    )(aggp, h0s, ni, no, W2)


def _tc_fin(aggp, t, ni):
    def body(a_r, t_r, ni_r, o_r):
        o_r[...] = (a_r[0] + a_r[1] + t_r[...]) * ni_r[...]

    return pl.pallas_call(
        body,
        out_shape=jax.ShapeDtypeStruct((NROWS, HID), jnp.float32),
    )(aggp, t, ni)


def kernel(x, edge_index, W1, W2):
    e = edge_index.shape[1]
    ncv = -(-e // (NW * CH))
    ncv = -(-ncv // 8) * 8
    e_pad = NW * ncv * CH
    pad = jnp.full((e_pad - e,), DUMMY, jnp.int32)
    src2d = jnp.concatenate([edge_index[0], pad]).reshape(e_pad // CH, CH)
    dst2d = jnp.concatenate([edge_index[1], pad]).reshape(e_pad // CH, CH)
    xp = jnp.pad(x, ((0, NROWS - x.shape[0]), (0, 0)))
    z1 = jnp.zeros((NROWS,), jnp.float32)
    z2 = jnp.zeros((NROWS, HID), jnp.float32)
    ones = jnp.ones((CH,), jnp.float32)

    degp_o, degp_i = _sc_degrees(src2d, dst2d, z1, ones)
    degp_o = degp_o.reshape(NC, NROWS)
    degp_i = degp_i.reshape(NC, NROWS)
    h0s, no, ni = _tc_prep(xp, W1, degp_o, degp_i)
    aggp1 = _sc_msgpass(h0s, src2d, dst2d, z2).reshape(NC, NROWS, HID)
    t = _tc_mid(aggp1, h0s, ni, no, W2)
    aggp2 = _sc_msgpass(t, src2d, dst2d, z2).reshape(NC, NROWS, HID)
    out = _tc_fin(aggp2, t, ni)
    return out[:N_NODES]
